# baseline (device time: 31600 ns/iter reference)
import jax
import jax.numpy as jnp
from jax import lax
from jax.experimental import pallas as pl
from jax.experimental.pallas import tpu as pltpu

N_DEV = 4
CAP = 136


def _broadcast_blocks(block):
    _, D = block.shape

    def body(block_ref, out_ref, send_sems, recv_sems):
        my = lax.axis_index("i")

        barrier = pltpu.get_barrier_semaphore()
        for p in range(N_DEV - 1):
            peer = (my + 1 + p) % N_DEV
            pl.semaphore_signal(
                barrier, inc=1, device_id=(peer,),
                device_id_type=pl.DeviceIdType.MESH,
            )
        pl.semaphore_wait(barrier, N_DEV - 1)

        rdmas = []
        for p in range(N_DEV - 1):
            peer = (my + 1 + p) % N_DEV
            rdma = pltpu.make_async_remote_copy(
                src_ref=block_ref.at[:, :],
                dst_ref=out_ref.at[pl.ds(my * CAP, CAP), :],
                send_sem=send_sems.at[p],
                recv_sem=recv_sems.at[2 - p],
                device_id=(peer,),
                device_id_type=pl.DeviceIdType.MESH,
            )
            rdma.start()
            rdmas.append(rdma)

        out_ref[pl.ds(my * CAP, CAP), :] = block_ref[:, :]

        for rdma in rdmas:
            rdma.wait()

    return pl.pallas_call(
        body,
        out_shape=jax.ShapeDtypeStruct((N_DEV * CAP, D), block.dtype),
        in_specs=[pl.BlockSpec(memory_space=pltpu.VMEM)],
        out_specs=pl.BlockSpec(memory_space=pltpu.VMEM),
        scratch_shapes=[
            pltpu.SemaphoreType.DMA((N_DEV - 1,)),
            pltpu.SemaphoreType.DMA((N_DEV - 1,)),
        ],
        compiler_params=pltpu.CompilerParams(collective_id=0),
    )(block)


def kernel(ids, E):
    T = ids.shape[0]
    V_per, _ = E.shape
    my = lax.axis_index("i")

    owner = ids // V_per
    order = jnp.argsort(owner, stable=True)
    ids_sorted = ids[order]
    owner_sorted = owner[order]
    counts = jnp.sum(owner[None, :] == jnp.arange(N_DEV)[:, None], axis=1)
    seg_off = jnp.concatenate(
        [jnp.zeros((1,), counts.dtype), jnp.cumsum(counts)[:-1]]
    )
    k = jnp.arange(T)
    slot_sorted = owner_sorted * CAP + (k - seg_off[owner_sorted])
    slot = jnp.zeros((T,), jnp.int32).at[order].set(slot_sorted.astype(jnp.int32))

    ids_sorted_pad = jnp.concatenate([ids_sorted, jnp.zeros((CAP,), ids.dtype)])
    my_ids = lax.dynamic_slice(ids_sorted_pad, (seg_off[my],), (CAP,))
    local = jnp.clip(my_ids - my * V_per, 0, V_per - 1)
    block = jnp.take(E, local, axis=0)

    slotbuf = _broadcast_blocks(block)
    return jnp.take(slotbuf, slot, axis=0)


# device time: 18586 ns/iter; 1.7002x vs baseline; 1.7002x over previous
import jax
import jax.numpy as jnp
from jax import lax
from jax.experimental import pallas as pl
from jax.experimental.pallas import tpu as pltpu

N_DEV = 4
CAP = 136


def _broadcast_blocks(block):
    _, D = block.shape

    def body(block_ref, out_ref, send_sems, recv_sems):
        my = lax.axis_index("i")

        barrier = pltpu.get_barrier_semaphore()
        for p in range(N_DEV - 1):
            peer = (my + 1 + p) % N_DEV
            pl.semaphore_signal(
                barrier, inc=1, device_id=(peer,),
                device_id_type=pl.DeviceIdType.MESH,
            )
        pl.semaphore_wait(barrier, N_DEV - 1)

        rdmas = []
        for p in range(N_DEV - 1):
            peer = (my + 1 + p) % N_DEV
            rdma = pltpu.make_async_remote_copy(
                src_ref=block_ref.at[:, :],
                dst_ref=out_ref.at[pl.ds(my * CAP, CAP), :],
                send_sem=send_sems.at[p],
                recv_sem=recv_sems.at[2 - p],
                device_id=(peer,),
                device_id_type=pl.DeviceIdType.MESH,
            )
            rdma.start()
            rdmas.append(rdma)

        out_ref[pl.ds(my * CAP, CAP), :] = block_ref[:, :]

        for rdma in rdmas:
            rdma.wait()

    return pl.pallas_call(
        body,
        out_shape=jax.ShapeDtypeStruct((N_DEV * CAP, D), block.dtype),
        in_specs=[pl.BlockSpec(memory_space=pltpu.VMEM)],
        out_specs=pl.BlockSpec(memory_space=pltpu.VMEM),
        scratch_shapes=[
            pltpu.SemaphoreType.DMA((N_DEV - 1,)),
            pltpu.SemaphoreType.DMA((N_DEV - 1,)),
        ],
        compiler_params=pltpu.CompilerParams(collective_id=0),
    )(block)


def kernel(ids, E):
    T = ids.shape[0]
    V_per, _ = E.shape
    my = lax.axis_index("i")
    bf16 = jnp.bfloat16

    owner = ids // V_per
    t_iota = jnp.arange(T, dtype=jnp.int32)
    same = owner[None, :] == owner[:, None]
    rank = jnp.sum(
        same & (t_iota[None, :] < t_iota[:, None]), axis=1, dtype=jnp.int32
    )
    slot = owner * CAP + rank

    mine = owner == my
    B = (
        (rank[None, :] == jnp.arange(CAP, dtype=jnp.int32)[:, None])
        & mine[None, :]
    )
    local = jnp.clip(ids - my * V_per, 0, V_per - 1).astype(jnp.float32)
    block_ids = jnp.dot(
        B.astype(jnp.float32), local[:, None],
        preferred_element_type=jnp.float32,
    ).astype(jnp.int32)[:, 0]

    O = (
        block_ids[:, None] == jnp.arange(V_per, dtype=jnp.int32)[None, :]
    ).astype(bf16)
    block = jnp.dot(O, E.astype(bf16), preferred_element_type=jnp.float32)

    slotbuf = _broadcast_blocks(block)

    P = (
        slot[:, None] == jnp.arange(N_DEV * CAP, dtype=jnp.int32)[None, :]
    ).astype(bf16)
    return jnp.dot(P, slotbuf.astype(bf16), preferred_element_type=jnp.float32)


# device time: 17441 ns/iter; 1.8118x vs baseline; 1.0656x over previous
import jax
import jax.numpy as jnp
from jax import lax
from jax.experimental import pallas as pl
from jax.experimental.pallas import tpu as pltpu

N_DEV = 4
CAP = 136


def _broadcast_blocks(block):
    _, D = block.shape

    def body(block_ref, out_ref, send_sems, recv_sems):
        my = lax.axis_index("i")

        barrier = pltpu.get_barrier_semaphore()
        for p in range(N_DEV - 1):
            peer = (my + 1 + p) % N_DEV
            pl.semaphore_signal(
                barrier, inc=1, device_id=(peer,),
                device_id_type=pl.DeviceIdType.MESH,
            )
        pl.semaphore_wait(barrier, N_DEV - 1)

        rdmas = []
        for p in range(N_DEV - 1):
            peer = (my + 1 + p) % N_DEV
            rdma = pltpu.make_async_remote_copy(
                src_ref=block_ref.at[:, :],
                dst_ref=out_ref.at[pl.ds(my * CAP, CAP), :],
                send_sem=send_sems.at[p],
                recv_sem=recv_sems.at[2 - p],
                device_id=(peer,),
                device_id_type=pl.DeviceIdType.MESH,
            )
            rdma.start()
            rdmas.append(rdma)

        out_ref[pl.ds(my * CAP, CAP), :] = block_ref[:, :]

        for rdma in rdmas:
            rdma.wait()

    return pl.pallas_call(
        body,
        out_shape=jax.ShapeDtypeStruct((N_DEV * CAP, D), block.dtype),
        in_specs=[pl.BlockSpec(memory_space=pltpu.VMEM)],
        out_specs=pl.BlockSpec(memory_space=pltpu.VMEM),
        scratch_shapes=[
            pltpu.SemaphoreType.DMA((N_DEV - 1,)),
            pltpu.SemaphoreType.DMA((N_DEV - 1,)),
        ],
        compiler_params=pltpu.CompilerParams(collective_id=0),
    )(block)


def kernel(ids, E):
    T = ids.shape[0]
    V_per, _ = E.shape
    my = lax.axis_index("i")
    bf16 = jnp.bfloat16

    owner = ids // V_per
    t_iota = jnp.arange(T, dtype=jnp.int32)
    same = owner[None, :] == owner[:, None]
    rank = jnp.sum(
        same & (t_iota[None, :] < t_iota[:, None]), axis=1, dtype=jnp.int32
    )
    slot = owner * CAP + rank

    mine = owner == my
    B = (
        (rank[None, :] == jnp.arange(CAP, dtype=jnp.int32)[:, None])
        & mine[None, :]
    )
    local = jnp.clip(ids - my * V_per, 0, V_per - 1).astype(jnp.float32)
    block_ids = jnp.dot(
        B.astype(jnp.float32), local[:, None],
        preferred_element_type=jnp.float32,
        precision=lax.Precision.HIGHEST,
    ).astype(jnp.int32)[:, 0]

    O = (
        block_ids[:, None] == jnp.arange(V_per, dtype=jnp.int32)[None, :]
    ).astype(bf16)
    block = jnp.dot(O, E.astype(bf16), preferred_element_type=jnp.float32)

    slotbuf = _broadcast_blocks(block)

    P = (
        slot[:, None] == jnp.arange(N_DEV * CAP, dtype=jnp.int32)[None, :]
    ).astype(bf16)
    return jnp.dot(P, slotbuf.astype(bf16), preferred_element_type=jnp.float32)


# device time: 16251 ns/iter; 1.9445x vs baseline; 1.0732x over previous
import jax
import jax.numpy as jnp
from jax import lax
from jax.experimental import pallas as pl
from jax.experimental.pallas import tpu as pltpu

N_DEV = 4
CAP = 136


def _broadcast_blocks(block):
    _, D = block.shape

    def body(block_ref, out_ref, send_sems, recv_sems):
        my = lax.axis_index("i")

        barrier = pltpu.get_barrier_semaphore()
        for p in range(N_DEV - 1):
            peer = (my + 1 + p) % N_DEV
            pl.semaphore_signal(
                barrier, inc=1, device_id=(peer,),
                device_id_type=pl.DeviceIdType.MESH,
            )
        pl.semaphore_wait(barrier, N_DEV - 1)

        rdmas = []
        for p in range(N_DEV - 1):
            peer = (my + 1 + p) % N_DEV
            rdma = pltpu.make_async_remote_copy(
                src_ref=block_ref.at[:, :],
                dst_ref=out_ref.at[pl.ds(my * CAP, CAP), :],
                send_sem=send_sems.at[p],
                recv_sem=recv_sems.at[2 - p],
                device_id=(peer,),
                device_id_type=pl.DeviceIdType.MESH,
            )
            rdma.start()
            rdmas.append(rdma)

        out_ref[pl.ds(my * CAP, CAP), :] = block_ref[:, :]

        for rdma in rdmas:
            rdma.wait()

    return pl.pallas_call(
        body,
        out_shape=jax.ShapeDtypeStruct((N_DEV * CAP, D), block.dtype),
        in_specs=[pl.BlockSpec(memory_space=pltpu.VMEM)],
        out_specs=pl.BlockSpec(memory_space=pltpu.VMEM),
        scratch_shapes=[
            pltpu.SemaphoreType.DMA((N_DEV - 1,)),
            pltpu.SemaphoreType.DMA((N_DEV - 1,)),
        ],
        compiler_params=pltpu.CompilerParams(collective_id=0),
    )(block)


def kernel(ids, E):
    T = ids.shape[0]
    V_per, _ = E.shape
    my = lax.axis_index("i")

    owner = ids // V_per
    t_iota = jnp.arange(T, dtype=jnp.int32)
    same = owner[None, :] == owner[:, None]
    rank = jnp.sum(
        same & (t_iota[None, :] < t_iota[:, None]), axis=1, dtype=jnp.int32
    )
    slot = owner * CAP + rank

    mine = owner == my
    B = (
        (rank[None, :] == jnp.arange(CAP, dtype=jnp.int32)[:, None])
        & mine[None, :]
    )
    local = jnp.bitwise_and(ids, V_per - 1)
    block_ids = jnp.sum(
        jnp.where(B, local[None, :], 0), axis=1, dtype=jnp.int32
    )

    block = jnp.take(E, block_ids, axis=0)
    slotbuf = _broadcast_blocks(block)
    return jnp.take(slotbuf, slot, axis=0)


# device time: 15272 ns/iter; 2.0691x vs baseline; 1.0641x over previous
import jax
import jax.numpy as jnp
from jax import lax
from jax.experimental import pallas as pl
from jax.experimental.pallas import tpu as pltpu

N_DEV = 4
CAP = 136


def _broadcast_blocks(block):
    _, D = block.shape

    def body(block_ref, out_ref, send_sems, recv_sems):
        my = lax.axis_index("i")

        barrier = pltpu.get_barrier_semaphore()
        for p in range(N_DEV - 1):
            peer = (my + 1 + p) % N_DEV
            pl.semaphore_signal(
                barrier, inc=1, device_id=(peer,),
                device_id_type=pl.DeviceIdType.MESH,
            )
        pl.semaphore_wait(barrier, N_DEV - 1)

        rdmas = []
        for p in range(N_DEV - 1):
            peer = (my + 1 + p) % N_DEV
            rdma = pltpu.make_async_remote_copy(
                src_ref=block_ref.at[:, :],
                dst_ref=out_ref.at[pl.ds(my * CAP, CAP), :],
                send_sem=send_sems.at[p],
                recv_sem=recv_sems.at[2 - p],
                device_id=(peer,),
                device_id_type=pl.DeviceIdType.MESH,
            )
            rdma.start()
            rdmas.append(rdma)

        out_ref[pl.ds(my * CAP, CAP), :] = block_ref[:, :]

        for rdma in rdmas:
            rdma.wait()

    return pl.pallas_call(
        body,
        out_shape=jax.ShapeDtypeStruct((N_DEV * CAP, D), block.dtype),
        in_specs=[pl.BlockSpec(memory_space=pltpu.VMEM)],
        out_specs=pl.BlockSpec(memory_space=pltpu.VMEM),
        scratch_shapes=[
            pltpu.SemaphoreType.DMA((N_DEV - 1,)),
            pltpu.SemaphoreType.DMA((N_DEV - 1,)),
        ],
        compiler_params=pltpu.CompilerParams(collective_id=0),
    )(block)


def kernel(ids, E):
    T = ids.shape[0]
    V_per, _ = E.shape
    my = lax.axis_index("i")

    owner = ids // V_per
    oh = (
        owner[:, None] == jnp.arange(N_DEV, dtype=jnp.int32)[None, :]
    ).astype(jnp.int32)
    incl = jnp.cumsum(oh, axis=0)
    rank = jnp.sum(jnp.where(oh != 0, incl - 1, 0), axis=1, dtype=jnp.int32)
    slot = owner * CAP + rank

    mine = owner == my
    B = (
        (rank[None, :] == jnp.arange(CAP, dtype=jnp.int32)[:, None])
        & mine[None, :]
    )
    local = jnp.bitwise_and(ids, V_per - 1)
    block_ids = jnp.sum(
        jnp.where(B, local[None, :], 0), axis=1, dtype=jnp.int32
    )

    block = jnp.take(E, block_ids, axis=0).astype(jnp.bfloat16)
    slotbuf = _broadcast_blocks(block)
    return jnp.take(slotbuf, slot, axis=0).astype(jnp.float32)


# device time: 12592 ns/iter; 2.5095x vs baseline; 1.2128x over previous
import jax
import jax.numpy as jnp
from jax import lax
from jax.experimental import pallas as pl
from jax.experimental.pallas import tpu as pltpu

N_DEV = 4
CAP = 136


def _index_math(ids_col, ids_row, v_per):
    T = ids_col.shape[0]

    def body(idc_ref, idr_ref, bid_ref, slot_ref):
        my = lax.axis_index("i")
        owner_c = idc_ref[:, :] // v_per
        owner_r = idr_ref[:, :] // v_per
        same = owner_c == owner_r
        i0 = lax.broadcasted_iota(jnp.int32, (T, T), 0)
        i1 = lax.broadcasted_iota(jnp.int32, (T, T), 1)
        earlier = (same & (i1 < i0)).astype(jnp.int32)
        rank_c = jnp.sum(earlier, axis=1, keepdims=True)
        later = (same & (i0 < i1)).astype(jnp.int32)
        rank_r = jnp.sum(later, axis=0, keepdims=True)
        slot_ref[:, :] = owner_c * CAP + rank_c

        jcap = lax.broadcasted_iota(jnp.int32, (CAP, T), 0)
        B = (jcap == rank_r) & (owner_r == my)
        local_r = idr_ref[:, :] & (v_per - 1)
        bid_ref[:, :] = jnp.sum(
            jnp.where(B, local_r, 0), axis=1, keepdims=True, dtype=jnp.int32
        )

    return pl.pallas_call(
        body,
        out_shape=(
            jax.ShapeDtypeStruct((CAP, 1), jnp.int32),
            jax.ShapeDtypeStruct((T, 1), jnp.int32),
        ),
        in_specs=[
            pl.BlockSpec(memory_space=pltpu.VMEM),
            pl.BlockSpec(memory_space=pltpu.VMEM),
        ],
        out_specs=(
            pl.BlockSpec(memory_space=pltpu.VMEM),
            pl.BlockSpec(memory_space=pltpu.VMEM),
        ),
    )(ids_col, ids_row)


def _broadcast_unpermute(block, slot, T):
    _, D = block.shape
    S = N_DEV * CAP

    def body(block_ref, slot_ref, out_ref, slotbuf, send_sems, recv_sems):
        my = lax.axis_index("i")

        barrier = pltpu.get_barrier_semaphore()
        for p in range(N_DEV - 1):
            peer = (my + 1 + p) % N_DEV
            pl.semaphore_signal(
                barrier, inc=1, device_id=(peer,),
                device_id_type=pl.DeviceIdType.MESH,
            )
        pl.semaphore_wait(barrier, N_DEV - 1)

        rdmas = []
        for p in range(N_DEV - 1):
            peer = (my + 1 + p) % N_DEV
            rdma = pltpu.make_async_remote_copy(
                src_ref=block_ref.at[:, :],
                dst_ref=slotbuf.at[pl.ds(my * CAP, CAP), :],
                send_sem=send_sems.at[p],
                recv_sem=recv_sems.at[2 - p],
                device_id=(peer,),
                device_id_type=pl.DeviceIdType.MESH,
            )
            rdma.start()
            rdmas.append(rdma)

        slotbuf[pl.ds(my * CAP, CAP), :] = block_ref[:, :]
        s_iota = lax.broadcasted_iota(jnp.int32, (T, S), 1)
        P = (slot_ref[:, :] == s_iota).astype(jnp.bfloat16)

        for rdma in rdmas:
            rdma.wait()

        out_ref[:, :] = jnp.dot(
            P, slotbuf[:, :], preferred_element_type=jnp.float32
        )

    return pl.pallas_call(
        body,
        out_shape=jax.ShapeDtypeStruct((T, D), jnp.float32),
        in_specs=[
            pl.BlockSpec(memory_space=pltpu.VMEM),
            pl.BlockSpec(memory_space=pltpu.VMEM),
        ],
        out_specs=pl.BlockSpec(memory_space=pltpu.VMEM),
        scratch_shapes=[
            pltpu.VMEM((S, D), jnp.bfloat16),
            pltpu.SemaphoreType.DMA((N_DEV - 1,)),
            pltpu.SemaphoreType.DMA((N_DEV - 1,)),
        ],
        compiler_params=pltpu.CompilerParams(collective_id=0),
    )(block, slot)


def kernel(ids, E):
    T = ids.shape[0]
    V_per, _ = E.shape

    block_ids, slot = _index_math(ids[:, None], ids[None, :], V_per)
    block = jnp.take(E, block_ids[:, 0], axis=0).astype(jnp.bfloat16)
    return _broadcast_unpermute(block, slot, T)


# device time: 11553 ns/iter; 2.7352x vs baseline; 1.0899x over previous
import jax
import jax.numpy as jnp
from jax import lax
from jax.experimental import pallas as pl
from jax.experimental.pallas import tpu as pltpu

N_DEV = 4
CAP = 136


def _index_math(ids_row, v_per):
    T = ids_row.shape[1]

    def body(idr_ref, bid_ref, slot_ref):
        my = lax.axis_index("i")
        owner_r = idr_ref[:, :] // v_per

        oh = lax.broadcasted_iota(jnp.int32, (N_DEV, T), 0) == owner_r
        i0 = lax.broadcasted_iota(jnp.int32, (T, T), 0)
        i1 = lax.broadcasted_iota(jnp.int32, (T, T), 1)
        lstrict = (i0 < i1).astype(jnp.bfloat16)
        C = jnp.dot(
            oh.astype(jnp.bfloat16), lstrict,
            preferred_element_type=jnp.float32,
        ).astype(jnp.int32)
        rank_r = jnp.sum(
            jnp.where(oh, C, 0), axis=0, keepdims=True, dtype=jnp.int32
        )
        slot_ref[:, :] = owner_r * CAP + rank_r

        jcap = lax.broadcasted_iota(jnp.int32, (CAP, T), 0)
        B = (jcap == rank_r) & (owner_r == my)
        local_r = idr_ref[:, :] & (v_per - 1)
        bid_ref[:, :] = jnp.sum(
            jnp.where(B, local_r, 0), axis=1, keepdims=True, dtype=jnp.int32
        )

    return pl.pallas_call(
        body,
        out_shape=(
            jax.ShapeDtypeStruct((CAP, 1), jnp.int32),
            jax.ShapeDtypeStruct((1, T), jnp.int32),
        ),
        in_specs=[pl.BlockSpec(memory_space=pltpu.VMEM)],
        out_specs=(
            pl.BlockSpec(memory_space=pltpu.VMEM),
            pl.BlockSpec(memory_space=pltpu.VMEM),
        ),
    )(ids_row)


def _broadcast_unpermute(block, slot, T):
    _, D = block.shape
    S = N_DEV * CAP

    def body(block_ref, slot_ref, out_ref, slotbuf, send_sems, recv_sems):
        my = lax.axis_index("i")

        barrier = pltpu.get_barrier_semaphore()
        for p in range(N_DEV - 1):
            peer = (my + 1 + p) % N_DEV
            pl.semaphore_signal(
                barrier, inc=1, device_id=(peer,),
                device_id_type=pl.DeviceIdType.MESH,
            )
        pl.semaphore_wait(barrier, N_DEV - 1)

        rdmas = []
        for p in range(N_DEV - 1):
            peer = (my + 1 + p) % N_DEV
            rdma = pltpu.make_async_remote_copy(
                src_ref=block_ref.at[:, :],
                dst_ref=slotbuf.at[pl.ds(my * CAP, CAP), :],
                send_sem=send_sems.at[p],
                recv_sem=recv_sems.at[2 - p],
                device_id=(peer,),
                device_id_type=pl.DeviceIdType.MESH,
            )
            rdma.start()
            rdmas.append(rdma)

        slotbuf[pl.ds(my * CAP, CAP), :] = block_ref[:, :]
        s_iota = lax.broadcasted_iota(jnp.int32, (S, T), 0)
        PT = (slot_ref[:, :] == s_iota).astype(jnp.bfloat16)

        for rdma in rdmas:
            rdma.wait()

        out_ref[:, :] = lax.dot_general(
            PT, slotbuf[:, :],
            dimension_numbers=(((0,), (0,)), ((), ())),
            preferred_element_type=jnp.float32,
        )

    return pl.pallas_call(
        body,
        out_shape=jax.ShapeDtypeStruct((T, D), jnp.float32),
        in_specs=[
            pl.BlockSpec(memory_space=pltpu.VMEM),
            pl.BlockSpec(memory_space=pltpu.VMEM),
        ],
        out_specs=pl.BlockSpec(memory_space=pltpu.VMEM),
        scratch_shapes=[
            pltpu.VMEM((S, D), jnp.bfloat16),
            pltpu.SemaphoreType.DMA((N_DEV - 1,)),
            pltpu.SemaphoreType.DMA((N_DEV - 1,)),
        ],
        compiler_params=pltpu.CompilerParams(collective_id=0),
    )(block, slot)


def kernel(ids, E):
    T = ids.shape[0]
    V_per, _ = E.shape

    block_ids, slot = _index_math(ids[None, :], V_per)
    block = jnp.take(E, block_ids[:, 0], axis=0).astype(jnp.bfloat16)
    return _broadcast_unpermute(block, slot, T)
